# baseline (speedup 1.0000x reference)
import functools

import jax
import jax.numpy as jnp
from jax import lax
from jax.experimental import pallas as pl
from jax.experimental.pallas import tpu as pltpu
from jax.experimental.pallas import tpu_sc as plsc

_N_SPARSE = 26
_N_DENSE = 13
_VOCAB = 100000
_EMB = 16
_B = 16384
_FLAT = _N_SPARSE * _EMB
_ROWS = _B * _N_SPARSE

_NC, _NS = 2, 16
_NW = _NC * _NS
_RPW = _ROWS // _NW
_CHUNK = 1024
_NCHUNK = _RPW // _CHUNK


def _sc_gather(tables_flat, idx_flat):
  mesh = plsc.VectorSubcoreMesh(core_axis_name="c", subcore_axis_name="s")

  @functools.partial(
      pl.kernel,
      mesh=mesh,
      out_type=jax.ShapeDtypeStruct((_ROWS, _EMB), jnp.float32),
      scratch_types=[
          pltpu.VMEM((_CHUNK,), jnp.int32),
          pltpu.VMEM((_CHUNK, _EMB), jnp.float32),
          pltpu.SemaphoreType.DMA,
      ],
      compiler_params=pltpu.CompilerParams(use_tc_tiling_on_sc=False),
  )
  def k(tab_hbm, idx_hbm, out_hbm, idx_v, rows_v, sem):
    wid = lax.axis_index("s") * _NC + lax.axis_index("c")
    base = wid * _RPW
    for j in range(_NCHUNK):
      off = base + j * _CHUNK
      pltpu.sync_copy(idx_hbm.at[pl.ds(off, _CHUNK)], idx_v)
      pltpu.async_copy(tab_hbm.at[idx_v], rows_v, sem).wait()
      pltpu.sync_copy(rows_v, out_hbm.at[pl.ds(off, _CHUNK)])

  return k(tables_flat, idx_flat)


def _tr_body(in_ref, out_ref):
  out_ref[...] = in_ref[0].T


def _tc_transpose(tphys):
  return pl.pallas_call(
      _tr_body,
      grid=(_N_SPARSE,),
      in_specs=[pl.BlockSpec((1, _EMB, _VOCAB), lambda f: (f, 0, 0))],
      out_specs=pl.BlockSpec((_VOCAB, _EMB), lambda f: (f, 0)),
      out_shape=jax.ShapeDtypeStruct((_N_SPARSE * _VOCAB, _EMB), jnp.float32),
      compiler_params=pltpu.CompilerParams(
          dimension_semantics=("arbitrary",)),
  )(tphys)


def _dnn_body(g_ref, d_ref, s_ref, w1a_ref, w1b_ref, b1_ref, w2_ref, b2_ref,
              w3_ref, b3_ref, wf_ref, bf_ref, out_ref):
  f32 = jnp.float32
  g = g_ref[...]
  dd = d_ref[...]
  sum_e = lax.dot(g, s_ref[...], preferred_element_type=f32)
  t1 = jnp.sum(sum_e * sum_e, axis=1, keepdims=True)
  t2 = jnp.sum(g * g, axis=1, keepdims=True)
  wide = 0.5 * (t1 - t2)
  h = lax.dot(g, w1a_ref[...], preferred_element_type=f32)
  h = h + lax.dot(dd, w1b_ref[...], preferred_element_type=f32)
  h = jax.nn.relu(h + b1_ref[...])
  h = jax.nn.relu(lax.dot(h, w2_ref[...], preferred_element_type=f32)
                  + b2_ref[...])
  h = jax.nn.relu(lax.dot(h, w3_ref[...], preferred_element_type=f32)
                  + b3_ref[...])
  z = lax.dot(wide + h, wf_ref[...], preferred_element_type=f32) + bf_ref[...]
  out_ref[...] = jax.nn.sigmoid(z)


_BB = 512


def _dnn(g, dense, s, w1a, w1b, b1, w2, b2, w3, b3, wf, bf):
  def row_block(i):
    return (i, 0)

  def full(i):
    return (0, 0)

  return pl.pallas_call(
      _dnn_body,
      grid=(_B // _BB,),
      in_specs=[
          pl.BlockSpec((_BB, _FLAT), row_block),
          pl.BlockSpec((_BB, _N_DENSE), row_block),
          pl.BlockSpec((_FLAT, _EMB), full),
          pl.BlockSpec((_FLAT, 256), full),
          pl.BlockSpec((_N_DENSE, 256), full),
          pl.BlockSpec((1, 256), full),
          pl.BlockSpec((256, 128), full),
          pl.BlockSpec((1, 128), full),
          pl.BlockSpec((128, 64), full),
          pl.BlockSpec((1, 64), full),
          pl.BlockSpec((64, 1), full),
          pl.BlockSpec((1, 1), full),
      ],
      out_specs=pl.BlockSpec((_BB, 1), row_block),
      out_shape=jax.ShapeDtypeStruct((_B, 1), jnp.float32),
      compiler_params=pltpu.CompilerParams(
          dimension_semantics=("parallel",)),
  )(g, dense, s, w1a, w1b, b1, w2, b2, w3, b3, wf, bf)


def kernel(x, tables, W1, b1, W2, b2, W3, b3, Wf, bf):
  sparse_idx = x[:, :_N_SPARSE].astype(jnp.int32)
  dense = x[:, _N_SPARSE:]
  field_off = (jnp.arange(_N_SPARSE, dtype=jnp.int32) * _VOCAB)[None, :]
  idx_flat = (sparse_idx + field_off).reshape(-1)
  tables_flat = tables.reshape(_N_SPARSE * _VOCAB, _EMB)
  gathered = _sc_gather(tables_flat, idx_flat)
  return gathered
  g = gathered.reshape(_B, _FLAT)
  s = jnp.tile(jnp.eye(_EMB, dtype=jnp.float32), (_N_SPARSE, 1))
  return _dnn(g, dense, s, W1[:_FLAT], W1[_FLAT:], b1.reshape(1, -1),
              W2, b2.reshape(1, -1), W3, b3.reshape(1, -1),
              Wf, bf.reshape(1, 1))


# speedup vs baseline: 4.5900x; 4.5900x over previous
import functools

import jax
import jax.numpy as jnp
from jax import lax
from jax.experimental import pallas as pl
from jax.experimental.pallas import tpu as pltpu
from jax.experimental.pallas import tpu_sc as plsc

_N_SPARSE = 26
_N_DENSE = 13
_VOCAB = 100000
_EMB = 16
_B = 16384
_FLAT = _N_SPARSE * _EMB
_ROWS = _B * _N_SPARSE

_NC, _NS = 2, 16
_NW = _NC * _NS
_RPW = _ROWS // _NW
_CHUNK = 1024
_NCHUNK = _RPW // _CHUNK


def _sc_gather(tables_flat, idx_flat):
  mesh = plsc.VectorSubcoreMesh(core_axis_name="c", subcore_axis_name="s")

  @functools.partial(
      pl.kernel,
      mesh=mesh,
      out_type=jax.ShapeDtypeStruct((_ROWS, _EMB), jnp.float32),
      scratch_types=[
          pltpu.VMEM((_CHUNK,), jnp.int32),
          pltpu.VMEM((_CHUNK, _EMB), jnp.float32),
          pltpu.SemaphoreType.DMA,
      ],
      compiler_params=pltpu.CompilerParams(use_tc_tiling_on_sc=False),
  )
  def k(tab_hbm, idx_hbm, out_hbm, idx_v, rows_v, sem):
    wid = lax.axis_index("s") * _NC + lax.axis_index("c")
    base = wid * _RPW
    for j in range(_NCHUNK):
      off = base + j * _CHUNK
      pltpu.sync_copy(idx_hbm.at[pl.ds(off, _CHUNK)], idx_v)
      pltpu.async_copy(tab_hbm.at[idx_v], rows_v, sem).wait()
      pltpu.sync_copy(rows_v, out_hbm.at[pl.ds(off, _CHUNK)])

  return k(tables_flat, idx_flat)


_VSEG = _VOCAB // 8


def _tr_body(in_ref, out_hbm, x_scr, y_scr, sem):
  f = pl.program_id(0)
  nf = pl.num_programs(0)
  for j in range(8):
    x_scr[j * _EMB:(j + 1) * _EMB, :] = in_ref[0, :, j * _VSEG:(j + 1) * _VSEG]
  off = jax.lax.rem(f, 2) * _VSEG

  @pl.when(f >= 2)
  def _wait_slot():
    pltpu.make_async_copy(
        y_scr.at[pl.ds(off, _VSEG)],
        out_hbm.at[pl.ds((f - 2) * _VSEG, _VSEG)], sem).wait()

  y_scr[pl.ds(off, _VSEG), :] = x_scr[...].T
  pltpu.make_async_copy(
      y_scr.at[pl.ds(off, _VSEG)],
      out_hbm.at[pl.ds(f * _VSEG, _VSEG)], sem).start()

  @pl.when(f == nf - 1)
  def _drain_all():
    for _ in range(2):
      pltpu.make_async_copy(
          y_scr.at[pl.ds(off, _VSEG)],
          out_hbm.at[pl.ds(f * _VSEG, _VSEG)], sem).wait()


def _tc_transpose(tphys):
  rows_pf = _VOCAB * _EMB // 128
  return pl.pallas_call(
      _tr_body,
      grid=(_N_SPARSE,),
      in_specs=[pl.BlockSpec((1, _EMB, _VOCAB), lambda f: (f, 0, 0))],
      out_specs=pl.BlockSpec(memory_space=pl.ANY),
      out_shape=jax.ShapeDtypeStruct((_N_SPARSE * rows_pf, 128), jnp.float32),
      scratch_shapes=[
          pltpu.VMEM((128, _VSEG), jnp.float32),
          pltpu.VMEM((2 * _VSEG, 128), jnp.float32),
          pltpu.SemaphoreType.DMA,
      ],
      compiler_params=pltpu.CompilerParams(
          dimension_semantics=("arbitrary",)),
  )(tphys)


def _dnn_body(g_ref, d_ref, s_ref, w1a_ref, w1b_ref, b1_ref, w2_ref, b2_ref,
              w3_ref, b3_ref, wf_ref, bf_ref, out_ref):
  f32 = jnp.float32
  g = g_ref[...]
  dd = d_ref[...]
  sum_e = lax.dot(g, s_ref[...], preferred_element_type=f32)
  t1 = jnp.sum(sum_e * sum_e, axis=1, keepdims=True)
  t2 = jnp.sum(g * g, axis=1, keepdims=True)
  wide = 0.5 * (t1 - t2)
  h = lax.dot(g, w1a_ref[...], preferred_element_type=f32)
  h = h + lax.dot(dd, w1b_ref[...], preferred_element_type=f32)
  h = jax.nn.relu(h + b1_ref[...])
  h = jax.nn.relu(lax.dot(h, w2_ref[...], preferred_element_type=f32)
                  + b2_ref[...])
  h = jax.nn.relu(lax.dot(h, w3_ref[...], preferred_element_type=f32)
                  + b3_ref[...])
  z = lax.dot(wide + h, wf_ref[...], preferred_element_type=f32) + bf_ref[...]
  out_ref[...] = jax.nn.sigmoid(z)


_BB = 512


def _dnn(g, dense, s, w1a, w1b, b1, w2, b2, w3, b3, wf, bf):
  def row_block(i):
    return (i, 0)

  def full(i):
    return (0, 0)

  return pl.pallas_call(
      _dnn_body,
      grid=(_B // _BB,),
      in_specs=[
          pl.BlockSpec((_BB, _FLAT), row_block),
          pl.BlockSpec((_BB, _N_DENSE), row_block),
          pl.BlockSpec((_FLAT, _EMB), full),
          pl.BlockSpec((_FLAT, 256), full),
          pl.BlockSpec((_N_DENSE, 256), full),
          pl.BlockSpec((1, 256), full),
          pl.BlockSpec((256, 128), full),
          pl.BlockSpec((1, 128), full),
          pl.BlockSpec((128, 64), full),
          pl.BlockSpec((1, 64), full),
          pl.BlockSpec((64, 1), full),
          pl.BlockSpec((1, 1), full),
      ],
      out_specs=pl.BlockSpec((_BB, 1), row_block),
      out_shape=jax.ShapeDtypeStruct((_B, 1), jnp.float32),
      compiler_params=pltpu.CompilerParams(
          dimension_semantics=("parallel",)),
  )(g, dense, s, w1a, w1b, b1, w2, b2, w3, b3, wf, bf)


def kernel(x, tables, W1, b1, W2, b2, W3, b3, Wf, bf):
  sparse_idx = x[:, :_N_SPARSE].astype(jnp.int32)
  dense = x[:, _N_SPARSE:]
  perm_row = (jnp.arange(_N_SPARSE, dtype=jnp.int32)[None, :] * _VSEG
              + sparse_idx % _VSEG) * 8 + sparse_idx // _VSEG
  idx_flat = perm_row.reshape(-1)
  tphys = jnp.transpose(tables, (0, 2, 1))
  t128 = _tc_transpose(tphys)
  tables_flat = t128.reshape(_N_SPARSE * _VOCAB, _EMB)
  gathered = _sc_gather(tables_flat, idx_flat)
  g = gathered.reshape(_B, _FLAT)
  s = jnp.tile(jnp.eye(_EMB, dtype=jnp.float32), (_N_SPARSE, 1))
  return _dnn(g, dense, s, W1[:_FLAT], W1[_FLAT:], b1.reshape(1, -1),
              W2, b2.reshape(1, -1), W3, b3.reshape(1, -1),
              Wf, bf.reshape(1, 1))
